# baseline (speedup 1.0000x reference)
import functools

import jax
import jax.numpy as jnp
from jax.experimental import pallas as pl
from jax.experimental.pallas import tpu as pltpu

B, N, T, F = 2, 207, 1024, 64
H = 8
PATCH = 8
TOPK = 4
FPH = F // H
TP = T // PATCH

_BF = jnp.bfloat16
_F32 = jnp.float32


def _split(a):
    hi = a.astype(_BF)
    lo = (a - hi.astype(_F32)).astype(_BF)
    return hi, lo


def _mm(ah, al, bh, bl, dn):
    def d(u, v):
        return jax.lax.dot_general(u, v, dn, preferred_element_type=_F32)

    return d(ah, bh) + d(ah, bl) + d(al, bh)


_DN_NN = (((1,), (0,)), ((), ()))
_DN_NT = (((1,), (1,)), ((), ()))
_DN_TT = (((0,), (1,)), ((), ()))
_DN_00 = (((0,), (0,)), ((), ()))


def _ac_kernel(x_ref, wqk_ref, wv_ref, wsum_ref, csvh_ref, csvl_ref,
               ceh_ref, cel_ref, g8_ref, e16_ref, o_ref):
    X = x_ref[0, 0].astype(_BF)
    CSvh = csvh_ref[...]
    CSvl = csvl_ref[...]

    qk = jax.lax.dot(X, wqk_ref[...], preferred_element_type=_F32)
    V2 = jax.lax.dot(X, wv_ref[...], preferred_element_type=_F32)

    qkh, qkl = _split(qk)
    A1 = _mm(CSvh, CSvl, qkh, qkl, _DN_NN)
    Qr = A1[0:TP, 0:H]
    Kr = A1[0:TP, H:2 * H]
    nQi = A1[TP:2 * TP, 0:H]
    nKi = A1[TP:2 * TP, H:2 * H]
    Pr = Qr * Kr + nQi * nKi
    nPi = nQi * Kr - Qr * nKi
    PPc = jnp.concatenate([Pr, nPi], axis=0)
    PPc = jnp.concatenate([PPc, PPc], axis=1)
    PPch, PPcl = _split(PPc)
    corrT = (_mm(PPch, PPcl, CSvh, CSvl, _DN_00)
             * (1.0 / TP))[0:H]

    lane = jax.lax.broadcasted_iota(jnp.int32, (H, TP), 1)
    work = corrT
    ws = []
    ds = []
    for _ in range(TOPK):
        m = jnp.max(work, axis=1, keepdims=True)
        idx = jnp.min(jnp.where(work == m, lane, TP), axis=1,
                      keepdims=True)
        ws.append(m)
        ds.append(idx)
        work = jnp.where(lane == idx, -jnp.inf, work)
    es = [jnp.exp(w - ws[0]) for w in ws]
    z = es[0] + es[1] + es[2] + es[3]

    s2T = jnp.zeros((H, TP), _F32)
    for e_i, d_i in zip(es, ds):
        s2T = s2T + jnp.where(lane == d_i, e_i / z, 0.0)

    s2h0, s2l0 = _split(s2T)
    s2h = jnp.concatenate([s2h0.T, s2h0.T], axis=1)
    s2l = jnp.concatenate([s2l0.T, s2l0.T], axis=1)
    A3 = _mm(CSvh, CSvl, s2h, s2l, _DN_NN)
    A3h, A3l = _split(A3)
    e = e16_ref[...]
    B3 = (jax.lax.dot_general(A3h, e, _DN_NN, preferred_element_type=_F32)
          + jax.lax.dot_general(A3l, e, _DN_NN,
                                preferred_element_type=_F32))
    SrB = B3[0:TP]
    nSiB = B3[TP:2 * TP]

    V2h, V2l = _split(V2)
    VF = _mm(CSvh, CSvl, V2h, V2l, _DN_NN)
    VFr = VF[0:TP]
    nVFi = VF[TP:2 * TP]
    Pr2 = VFr * SrB + nVFi * nSiB
    nPi2 = nVFi * SrB - VFr * nSiB
    PP2 = jnp.concatenate([Pr2, nPi2], axis=0)

    PP2h, PP2l = _split(PP2)
    g = g8_ref[...]
    PPG = (jax.lax.dot_general(PP2h, g, _DN_NN, preferred_element_type=_F32)
           + jax.lax.dot_general(PP2l, g, _DN_NN,
                                 preferred_element_type=_F32))
    PPGh, PPGl = _split(PPG)
    AGGfull = (_mm(PPGh, PPGl, ceh_ref[...], cel_ref[...],
                   _DN_00) * (1.0 / TP))[0:PATCH]
    li = jax.lax.broadcasted_iota(jnp.int32, (PATCH, T), 1)
    ri = jax.lax.broadcasted_iota(jnp.int32, (PATCH, T), 0)
    agg = jnp.sum(jnp.where(li % PATCH == ri, AGGfull, 0.0),
                  axis=0, keepdims=True)
    agg = agg.astype(_BF).astype(_F32)
    o_ref[0, 0] = wsum_ref[...] * agg


@functools.partial(jax.jit, static_argnames=("interpret",))
def kernel(x, Wq, Wk, Wv, Wout, interpret=False):
    xr = x.reshape(B, N, TP, PATCH * F)
    Wq2 = Wq.transpose(2, 1, 0).reshape(PATCH * F, H)
    Wk2 = Wk.transpose(2, 1, 0).reshape(PATCH * F, H)
    Wqk = jnp.concatenate([Wq2, Wk2], axis=1).astype(_BF)
    Wvblk = jnp.kron(jnp.eye(PATCH, dtype=_F32), Wv.T).astype(_BF)
    Wsum = (Wout.astype(_BF).astype(_F32).sum(axis=1).reshape(F, 1))
    idx = jnp.arange(TP, dtype=_F32)
    ang = (2.0 * jnp.pi / TP) * (idx[:, None] * idx[None, :])
    CSv = jnp.concatenate([jnp.cos(ang), jnp.sin(ang)], axis=0)
    CSvh = CSv.astype(_BF)
    CSvl = (CSv - CSvh.astype(_F32)).astype(_BF)
    CSexp = jnp.repeat(CSv, PATCH, axis=1)
    CSexph = CSexp.astype(_BF)
    CSexpl = (CSexp - CSexph.astype(_F32)).astype(_BF)
    eyeP = jnp.eye(PATCH, dtype=_F32)
    G8 = jnp.broadcast_to(eyeP[:, :, None], (PATCH, PATCH, H)).reshape(PATCH, F)
    G8 = (G8.T * (1.0 / H))
    G8 = jnp.concatenate([G8, G8], axis=1).astype(_BF)
    eyeH = jnp.eye(H, dtype=_F32)
    E = jnp.broadcast_to(eyeH[:, None, :], (H, PATCH, H)).reshape(H, F)
    E16 = jnp.concatenate([E, jnp.zeros((H, F), _F32)], axis=0).astype(_BF)

    out = pl.pallas_call(
        _ac_kernel,
        grid=(B, N),
        in_specs=[
            pl.BlockSpec((1, 1, TP, PATCH * F), lambda b, n: (b, n, 0, 0)),
            pl.BlockSpec((PATCH * F, 2 * H), lambda b, n: (0, 0)),
            pl.BlockSpec((PATCH * F, F), lambda b, n: (0, 0)),
            pl.BlockSpec((F, 1), lambda b, n: (0, 0)),
            pl.BlockSpec((2 * TP, TP), lambda b, n: (0, 0)),
            pl.BlockSpec((2 * TP, TP), lambda b, n: (0, 0)),
            pl.BlockSpec((2 * TP, T), lambda b, n: (0, 0)),
            pl.BlockSpec((2 * TP, T), lambda b, n: (0, 0)),
            pl.BlockSpec((F, 2 * H), lambda b, n: (0, 0)),
            pl.BlockSpec((2 * H, F), lambda b, n: (0, 0)),
        ],
        out_specs=pl.BlockSpec((1, 1, F, T), lambda b, n: (b, n, 0, 0)),
        out_shape=jax.ShapeDtypeStruct((B, N, F, T), jnp.float32),
        compiler_params=pltpu.CompilerParams(
            dimension_semantics=("parallel", "parallel"),
        ),
        interpret=interpret,
    )(xr, Wqk, Wvblk, Wsum, CSvh, CSvl, CSexph, CSexpl, G8, E16)
    return out


# speedup vs baseline: 1.3803x; 1.3803x over previous
import functools

import jax
import jax.numpy as jnp
from jax.experimental import pallas as pl
from jax.experimental.pallas import tpu as pltpu

B, N, T, F = 2, 207, 1024, 64
H = 8
PATCH = 8
TOPK = 4
FPH = F // H
TP = T // PATCH

def _split(a):
    hi = a.astype(jnp.bfloat16)
    lo = (a - hi.astype(jnp.float32)).astype(jnp.bfloat16)
    return hi, lo


def _dot(a, b):
    ah, al = _split(a)
    bh, bl = _split(b)

    def d(u, v):
        return jax.lax.dot(u, v, preferred_element_type=jnp.float32)

    return d(ah, bh) + d(ah, bl) + d(al, bh)


def _dot_bf16(a, b):
    return jax.lax.dot(a.astype(jnp.bfloat16), b.astype(jnp.bfloat16),
                       preferred_element_type=jnp.float32)


def _dote(a, e):
    ah, al = _split(a)
    return (jax.lax.dot(ah, e, preferred_element_type=jnp.float32)
            + jax.lax.dot(al, e, preferred_element_type=jnp.float32))


def _ac_kernel(x_ref, wq_ref, wk_ref, wv_ref, wsum_ref, c_ref, s_ref,
               t8_ref, o_ref):
    X = x_ref[0, 0]
    C = c_ref[...]
    S = s_ref[...]

    q = _dot_bf16(X, wq_ref[...])
    k = _dot_bf16(X, wk_ref[...])
    V2 = _dot_bf16(X, wv_ref[...])

    Qr = _dot(C, q)
    Qi = -_dot(S, q)
    Kr = _dot(C, k)
    Ki = -_dot(S, k)
    Pr = Qr * Kr + Qi * Ki
    Pi = Qi * Kr - Qr * Ki
    corr = (_dot(C, Pr) - _dot(S, Pi)) * (1.0 / TP)

    rowi = jax.lax.broadcasted_iota(jnp.int32, (TP, H), 0)
    work = corr
    ws = []
    ds = []
    for _ in range(TOPK):
        m = jnp.max(work, axis=0, keepdims=True)
        idx = jnp.min(jnp.where(work == m, rowi, TP), axis=0,
                      keepdims=True)
        ws.append(m)
        ds.append(idx)
        work = jnp.where(rowi == idx, -jnp.inf, work)
    mx = ws[0]
    es = [jnp.exp(w - mx) for w in ws]
    z = es[0] + es[1] + es[2] + es[3]
    sms = [e / z for e in es]

    s2 = jnp.zeros((TP, H), jnp.float32)
    for sm_i, d_i in zip(sms, ds):
        s2 = s2 + jnp.where(rowi == d_i, sm_i, 0.0)

    Sr = _dot(C, s2)
    Si = -_dot(S, s2)
    srB = _dote(Sr, t8_ref[...])
    siB = _dote(Si, t8_ref[...])
    VFr = _dot(C, V2)
    VFi = -_dot(S, V2)
    Pr2 = VFr * srB + VFi * siB
    Pi2 = VFi * srB - VFr * siB
    aggr = (_dot(C, Pr2) - _dot(S, Pi2)) * (1.0 / TP)
    aggm = aggr.reshape(TP, PATCH, H).sum(axis=2) * (1.0 / H)
    agg = aggm.reshape(1, T)
    agg = agg.astype(jnp.bfloat16).astype(jnp.float32)
    o_ref[0, 0] = wsum_ref[...] * agg


@functools.partial(jax.jit, static_argnames=("interpret",))
def kernel(x, Wq, Wk, Wv, Wout, interpret=False):
    xr = x.reshape(B, N, TP, PATCH * F)
    Wq2 = Wq.transpose(2, 1, 0).reshape(PATCH * F, H)
    Wk2 = Wk.transpose(2, 1, 0).reshape(PATCH * F, H)
    Wvblk = jnp.kron(jnp.eye(PATCH, dtype=jnp.float32), Wv.T)
    Wsum = (Wout.astype(jnp.bfloat16).astype(jnp.float32)
            .sum(axis=1).reshape(F, 1))
    idx = jnp.arange(TP, dtype=jnp.float32)
    ang = (2.0 * jnp.pi / TP) * (idx[:, None] * idx[None, :])
    C = jnp.cos(ang)
    S = jnp.sin(ang)
    T8 = jnp.tile(jnp.eye(H, dtype=jnp.float32),
                  (1, PATCH)).astype(jnp.bfloat16)

    const = pl.BlockSpec(None, lambda b, n: (0, 0))
    out = pl.pallas_call(
        _ac_kernel,
        grid=(B, N),
        in_specs=[
            pl.BlockSpec((1, 1, TP, PATCH * F), lambda b, n: (b, n, 0, 0)),
            pl.BlockSpec((PATCH * F, H), lambda b, n: (0, 0)),
            pl.BlockSpec((PATCH * F, H), lambda b, n: (0, 0)),
            pl.BlockSpec((PATCH * F, F), lambda b, n: (0, 0)),
            pl.BlockSpec((F, 1), lambda b, n: (0, 0)),
            pl.BlockSpec((TP, TP), lambda b, n: (0, 0)),
            pl.BlockSpec((TP, TP), lambda b, n: (0, 0)),
            pl.BlockSpec((H, F), lambda b, n: (0, 0)),
        ],
        out_specs=pl.BlockSpec((1, 1, F, T), lambda b, n: (b, n, 0, 0)),
        out_shape=jax.ShapeDtypeStruct((B, N, F, T), jnp.float32),
        compiler_params=pltpu.CompilerParams(
            dimension_semantics=("parallel", "parallel"),
        ),
        interpret=interpret,
    )(xr, Wq2, Wk2, Wvblk, Wsum, C, S, T8)
    return out


# speedup vs baseline: 1.5782x; 1.1434x over previous
import functools

import jax
import jax.numpy as jnp
from jax.experimental import pallas as pl
from jax.experimental.pallas import tpu as pltpu

B, N, T, F = 2, 207, 1024, 64
H = 8
PATCH = 8
TOPK = 4
FPH = F // H
TP = T // PATCH
NTILE = 3

def _split(a):
    hi = a.astype(jnp.bfloat16)
    lo = (a - hi.astype(jnp.float32)).astype(jnp.bfloat16)
    return hi, lo


def _dot(a, b):
    ah, al = _split(a)
    bh, bl = _split(b)

    def d(u, v):
        return jax.lax.dot(u, v, preferred_element_type=jnp.float32)

    return d(ah, bh) + d(ah, bl) + d(al, bh)


def _dot_bf16(a, b):
    return jax.lax.dot(a.astype(jnp.bfloat16), b.astype(jnp.bfloat16),
                       preferred_element_type=jnp.float32)


def _dote(a, e):
    ah, al = _split(a)
    return (jax.lax.dot(ah, e, preferred_element_type=jnp.float32)
            + jax.lax.dot(al, e, preferred_element_type=jnp.float32))


def _ac_kernel(x_ref, wq_ref, wk_ref, wv_ref, wsum_ref, c_ref, s_ref,
               t8_ref, o_ref):
    C = c_ref[...]
    S = s_ref[...]
    for t in range(NTILE):
        _one_tile(x_ref[0, t], wq_ref, wk_ref, wv_ref, wsum_ref, C, S,
                  t8_ref, o_ref, t)


def _one_tile(X, wq_ref, wk_ref, wv_ref, wsum_ref, C, S, t8_ref, o_ref, t):
    q = _dot_bf16(X, wq_ref[...])
    k = _dot_bf16(X, wk_ref[...])
    V2 = _dot_bf16(X, wv_ref[...])

    Qr = _dot(C, q)
    Qi = -_dot(S, q)
    Kr = _dot(C, k)
    Ki = -_dot(S, k)
    Pr = Qr * Kr + Qi * Ki
    Pi = Qi * Kr - Qr * Ki
    corr = (_dot(C, Pr) - _dot(S, Pi)) * (1.0 / TP)

    rowi = jax.lax.broadcasted_iota(jnp.int32, (TP, H), 0)
    work = corr
    ws = []
    ds = []
    for _ in range(TOPK):
        m = jnp.max(work, axis=0, keepdims=True)
        idx = jnp.min(jnp.where(work == m, rowi, TP), axis=0,
                      keepdims=True)
        ws.append(m)
        ds.append(idx)
        work = jnp.where(rowi == idx, -jnp.inf, work)
    mx = ws[0]
    es = [jnp.exp(w - mx) for w in ws]
    z = es[0] + es[1] + es[2] + es[3]
    sms = [e / z for e in es]

    s2 = jnp.zeros((TP, H), jnp.float32)
    for sm_i, d_i in zip(sms, ds):
        s2 = s2 + jnp.where(rowi == d_i, sm_i, 0.0)

    Sr = _dot(C, s2)
    Si = -_dot(S, s2)
    srB = _dote(Sr, t8_ref[...])
    siB = _dote(Si, t8_ref[...])
    VFr = _dot(C, V2)
    VFi = -_dot(S, V2)
    Pr2 = VFr * srB + VFi * siB
    Pi2 = VFi * srB - VFr * siB
    aggr = (_dot(C, Pr2) - _dot(S, Pi2)) * (1.0 / TP)
    aggm = aggr.reshape(TP, PATCH, H).sum(axis=2) * (1.0 / H)
    agg = aggm.reshape(1, T)
    agg = agg.astype(jnp.bfloat16).astype(jnp.float32)
    o_ref[0, t] = wsum_ref[...] * agg


@functools.partial(jax.jit, static_argnames=("interpret",))
def kernel(x, Wq, Wk, Wv, Wout, interpret=False):
    xr = x.reshape(B, N, TP, PATCH * F)
    Wq2 = Wq.transpose(2, 1, 0).reshape(PATCH * F, H)
    Wk2 = Wk.transpose(2, 1, 0).reshape(PATCH * F, H)
    Wvblk = jnp.kron(jnp.eye(PATCH, dtype=jnp.float32), Wv.T)
    Wsum = (Wout.astype(jnp.bfloat16).astype(jnp.float32)
            .sum(axis=1).reshape(F, 1))
    idx = jnp.arange(TP, dtype=jnp.float32)
    ang = (2.0 * jnp.pi / TP) * (idx[:, None] * idx[None, :])
    C = jnp.cos(ang)
    S = jnp.sin(ang)
    T8 = jnp.tile(jnp.eye(H, dtype=jnp.float32),
                  (1, PATCH)).astype(jnp.bfloat16)

    const = pl.BlockSpec(None, lambda b, n: (0, 0))
    out = pl.pallas_call(
        _ac_kernel,
        grid=(B, N // NTILE),
        in_specs=[
            pl.BlockSpec((1, NTILE, TP, PATCH * F), lambda b, n: (b, n, 0, 0)),
            pl.BlockSpec((PATCH * F, H), lambda b, n: (0, 0)),
            pl.BlockSpec((PATCH * F, H), lambda b, n: (0, 0)),
            pl.BlockSpec((PATCH * F, F), lambda b, n: (0, 0)),
            pl.BlockSpec((F, 1), lambda b, n: (0, 0)),
            pl.BlockSpec((TP, TP), lambda b, n: (0, 0)),
            pl.BlockSpec((TP, TP), lambda b, n: (0, 0)),
            pl.BlockSpec((H, F), lambda b, n: (0, 0)),
        ],
        out_specs=pl.BlockSpec((1, NTILE, F, T), lambda b, n: (b, n, 0, 0)),
        out_shape=jax.ShapeDtypeStruct((B, N, F, T), jnp.float32),
        compiler_params=pltpu.CompilerParams(
            dimension_semantics=("parallel", "parallel"),
        ),
        interpret=interpret,
    )(xr, Wq2, Wk2, Wvblk, Wsum, C, S, T8)
    return out


# speedup vs baseline: 1.6636x; 1.0541x over previous
import functools

import jax
import jax.numpy as jnp
from jax.experimental import pallas as pl
from jax.experimental.pallas import tpu as pltpu

B, N, T, F = 2, 207, 1024, 64
H = 8
PATCH = 8
TOPK = 4
FPH = F // H
TP = T // PATCH
NTILE = 9

def _split(a):
    hi = a.astype(jnp.bfloat16)
    lo = (a - hi.astype(jnp.float32)).astype(jnp.bfloat16)
    return hi, lo


def _dot(a, b):
    ah, al = _split(a)
    bh, bl = _split(b)

    def d(u, v):
        return jax.lax.dot(u, v, preferred_element_type=jnp.float32)

    return d(ah, bh) + d(ah, bl) + d(al, bh)


def _dot_bf16(a, b):
    return jax.lax.dot(a.astype(jnp.bfloat16), b.astype(jnp.bfloat16),
                       preferred_element_type=jnp.float32)


def _dote(a, e):
    ah, al = _split(a)
    return (jax.lax.dot(ah, e, preferred_element_type=jnp.float32)
            + jax.lax.dot(al, e, preferred_element_type=jnp.float32))


def _ac_kernel(x_ref, wq_ref, wk_ref, wv_ref, wsum_ref, c_ref, s_ref,
               t8_ref, o_ref):
    C = c_ref[...]
    S = s_ref[...]
    for t in range(NTILE):
        _one_tile(x_ref[0, t], wq_ref, wk_ref, wv_ref, wsum_ref, C, S,
                  t8_ref, o_ref, t)


def _one_tile(X, wq_ref, wk_ref, wv_ref, wsum_ref, C, S, t8_ref, o_ref, t):
    q = _dot_bf16(X, wq_ref[...])
    k = _dot_bf16(X, wk_ref[...])
    V2 = _dot_bf16(X, wv_ref[...])

    Qr = _dot(C, q)
    Qi = -_dot(S, q)
    Kr = _dot(C, k)
    Ki = -_dot(S, k)
    Pr = Qr * Kr + Qi * Ki
    Pi = Qi * Kr - Qr * Ki
    corr = (_dot(C, Pr) - _dot(S, Pi)) * (1.0 / TP)

    rowi = jax.lax.broadcasted_iota(jnp.int32, (TP, H), 0)
    work = corr
    ws = []
    ds = []
    for _ in range(TOPK):
        m = jnp.max(work, axis=0, keepdims=True)
        idx = jnp.min(jnp.where(work == m, rowi, TP), axis=0,
                      keepdims=True)
        ws.append(m)
        ds.append(idx)
        work = jnp.where(rowi == idx, -jnp.inf, work)
    mx = ws[0]
    es = [jnp.exp(w - mx) for w in ws]
    z = es[0] + es[1] + es[2] + es[3]
    sms = [e / z for e in es]

    s2 = jnp.zeros((TP, H), jnp.float32)
    for sm_i, d_i in zip(sms, ds):
        s2 = s2 + jnp.where(rowi == d_i, sm_i, 0.0)

    Sr = _dot(C, s2)
    Si = -_dot(S, s2)
    srB = _dote(Sr, t8_ref[...])
    siB = _dote(Si, t8_ref[...])
    VFr = _dot(C, V2)
    VFi = -_dot(S, V2)
    Pr2 = VFr * srB + VFi * siB
    Pi2 = VFi * srB - VFr * siB
    aggr = (_dot(C, Pr2) - _dot(S, Pi2)) * (1.0 / TP)
    aggm = aggr.reshape(TP, PATCH, H).sum(axis=2) * (1.0 / H)
    agg = aggm.reshape(1, T)
    agg = agg.astype(jnp.bfloat16).astype(jnp.float32)
    o_ref[0, t] = wsum_ref[...] * agg


@functools.partial(jax.jit, static_argnames=("interpret",))
def kernel(x, Wq, Wk, Wv, Wout, interpret=False):
    xr = x.reshape(B, N, TP, PATCH * F)
    Wq2 = Wq.transpose(2, 1, 0).reshape(PATCH * F, H)
    Wk2 = Wk.transpose(2, 1, 0).reshape(PATCH * F, H)
    Wvblk = jnp.kron(jnp.eye(PATCH, dtype=jnp.float32), Wv.T)
    Wsum = (Wout.astype(jnp.bfloat16).astype(jnp.float32)
            .sum(axis=1).reshape(F, 1))
    idx = jnp.arange(TP, dtype=jnp.float32)
    ang = (2.0 * jnp.pi / TP) * (idx[:, None] * idx[None, :])
    C = jnp.cos(ang)
    S = jnp.sin(ang)
    T8 = jnp.tile(jnp.eye(H, dtype=jnp.float32),
                  (1, PATCH)).astype(jnp.bfloat16)

    const = pl.BlockSpec(None, lambda b, n: (0, 0))
    out = pl.pallas_call(
        _ac_kernel,
        grid=(B, N // NTILE),
        in_specs=[
            pl.BlockSpec((1, NTILE, TP, PATCH * F), lambda b, n: (b, n, 0, 0)),
            pl.BlockSpec((PATCH * F, H), lambda b, n: (0, 0)),
            pl.BlockSpec((PATCH * F, H), lambda b, n: (0, 0)),
            pl.BlockSpec((PATCH * F, F), lambda b, n: (0, 0)),
            pl.BlockSpec((F, 1), lambda b, n: (0, 0)),
            pl.BlockSpec((TP, TP), lambda b, n: (0, 0)),
            pl.BlockSpec((TP, TP), lambda b, n: (0, 0)),
            pl.BlockSpec((H, F), lambda b, n: (0, 0)),
        ],
        out_specs=pl.BlockSpec((1, NTILE, F, T), lambda b, n: (b, n, 0, 0)),
        out_shape=jax.ShapeDtypeStruct((B, N, F, T), jnp.float32),
        compiler_params=pltpu.CompilerParams(
            dimension_semantics=("parallel", "parallel"),
        ),
        interpret=interpret,
    )(xr, Wq2, Wk2, Wvblk, Wsum, C, S, T8)
    return out
